# baseline (speedup 1.0000x reference)
import functools

import jax
import jax.numpy as jnp
from jax import lax
from jax.experimental import pallas as pl
from jax.experimental.pallas import tpu as pltpu
from jax.experimental.pallas import tpu_sc as plsc

VOCAB = 1000000
EMBED = 64
B = 4096
L = 200
LANES = 128

NC = 2
NS = 16
NW = NC * NS
NIDX = (B * L) // LANES
CPW = NIDX // NW

A_BLK = 512
A_GRID = (VOCAB + A_BLK - 1) // A_BLK


def _widen_body(wt_ref, wide_ref):
    wide_ref[:, :EMBED] = wt_ref[...].T


def _widen(W_E):
    wt = W_E.T
    return pl.pallas_call(
        _widen_body,
        grid=(A_GRID,),
        in_specs=[pl.BlockSpec((EMBED, A_BLK), lambda i: (0, i))],
        out_specs=pl.BlockSpec((A_BLK, LANES), lambda i: (i, 0)),
        out_shape=jax.ShapeDtypeStruct((VOCAB, LANES), jnp.float32),
        compiler_params=pltpu.CompilerParams(
            dimension_semantics=("arbitrary",)),
    )(wt)


def _worker_body(tok_hbm, wE_hbm, pos_hbm, out_hbm,
                 idx_v, pos_v, g0, g1, o0, o1, gs0, gs1, ws0, ws1):
    cid = lax.axis_index("c")
    sid = lax.axis_index("s")
    wid = sid * NC + cid
    base = wid * CPW

    pltpu.sync_copy(tok_hbm.at[pl.ds(base, CPW)], idx_v)
    pltpu.sync_copy(pos_hbm, pos_v)

    gbuf = (g0, g1)
    obuf = (o0, o1)
    gsem = (gs0, gs1)
    wsem = (ws0, ws1)

    def gather_copy(cc, b):
        return pltpu.make_async_copy(wE_hbm.at[idx_v.at[cc]], gbuf[b], gsem[b])

    def write_copy(cc, b):
        return pltpu.make_async_copy(obuf[b], out_hbm.at[base + cc], wsem[b])

    def chunk_step(cc, b):
        gather_copy(cc, b).wait()

        @pl.when(cc >= 2)
        def _():
            write_copy(cc - 2, b).wait()

        start = lax.rem(cc * LANES, L)
        split = L - start

        def add_lo(j, carry):
            for k in range(EMBED // 16):
                sl = pl.ds(16 * k, 16)
                obuf[b][j, sl] = gbuf[b][j, sl] + pos_v[start + j, sl]
            return carry

        def add_hi(j, carry):
            for k in range(EMBED // 16):
                sl = pl.ds(16 * k, 16)
                obuf[b][j, sl] = gbuf[b][j, sl] + pos_v[start + j - L, sl]
            return carry

        lax.fori_loop(0, lax.min(split, LANES), add_lo, 0)
        lax.fori_loop(lax.min(split, LANES), LANES, add_hi, 0)

        @pl.when(cc + 2 < CPW)
        def _():
            gather_copy(cc + 2, b).start()

        write_copy(cc, b).start()

    gather_copy(0, 0).start()
    gather_copy(1, 1).start()

    def loop_body(i, carry):
        chunk_step(2 * i, 0)
        chunk_step(2 * i + 1, 1)
        return carry

    lax.fori_loop(0, CPW // 2, loop_body, 0)

    write_copy(CPW - 2, 0).wait()
    write_copy(CPW - 1, 1).wait()


def _sc_embed(tok, W_E_wide, W_pos):
    mesh = plsc.VectorSubcoreMesh(core_axis_name="c", subcore_axis_name="s")
    kern = functools.partial(
        pl.kernel,
        out_type=jax.ShapeDtypeStruct((NIDX, LANES, EMBED), jnp.float32),
        mesh=mesh,
        scratch_types=[
            pltpu.VMEM((CPW, LANES), jnp.int32),
            pltpu.VMEM((L, EMBED), jnp.float32),
            pltpu.VMEM((LANES, LANES), jnp.float32),
            pltpu.VMEM((LANES, LANES), jnp.float32),
            pltpu.VMEM((LANES, EMBED), jnp.float32),
            pltpu.VMEM((LANES, EMBED), jnp.float32),
            pltpu.SemaphoreType.DMA,
            pltpu.SemaphoreType.DMA,
            pltpu.SemaphoreType.DMA,
            pltpu.SemaphoreType.DMA,
        ],
        compiler_params=pltpu.CompilerParams(use_tc_tiling_on_sc=True),
    )(_worker_body)
    return kern(tok, W_E_wide, W_pos)


def kernel(tokens, W_E, W_pos):
    tok = tokens.reshape(NIDX, LANES).astype(jnp.int32)
    wide = _widen(W_E)
    out = _sc_embed(tok, wide, W_pos)
    return out.reshape(B, L, EMBED)


# speedup vs baseline: 1.6979x; 1.6979x over previous
import functools

import jax
import jax.numpy as jnp
from jax import lax
from jax.experimental import pallas as pl
from jax.experimental.pallas import tpu as pltpu
from jax.experimental.pallas import tpu_sc as plsc

VOCAB = 1000000
EMBED = 64
B = 4096
L = 200
LANES = 128

NC = 2
NS = 16
NW = NC * NS
NIDX = (B * L) // LANES
CPW = NIDX // NW

A_BLK = 2048
A_GRID = (VOCAB + A_BLK - 1) // A_BLK


def _widen_body(wt_ref, wide_ref):
    eye = jnp.eye(EMBED, LANES, dtype=jnp.float32)
    wide_ref[...] = jax.lax.dot_general(
        wt_ref[...], eye, (((0,), (0,)), ((), ())),
        preferred_element_type=jnp.float32)


def _widen(W_E):
    wt = W_E.T
    return pl.pallas_call(
        _widen_body,
        grid=(A_GRID,),
        in_specs=[pl.BlockSpec((EMBED, A_BLK), lambda i: (0, i))],
        out_specs=pl.BlockSpec((A_BLK, LANES), lambda i: (i, 0)),
        out_shape=jax.ShapeDtypeStruct((VOCAB, LANES), jnp.float32),
        compiler_params=pltpu.CompilerParams(
            dimension_semantics=("arbitrary",)),
    )(wt)


def _worker_body(tok_hbm, wE_hbm, pos_hbm, out_hbm,
                 idx_v, pos_v, g0, g1, o0, o1, gs0, gs1, ws0, ws1):
    cid = lax.axis_index("c")
    sid = lax.axis_index("s")
    wid = sid * NC + cid
    base = wid * CPW

    pltpu.sync_copy(tok_hbm.at[pl.ds(base, CPW)], idx_v)
    pltpu.sync_copy(pos_hbm, pos_v.at[pl.ds(0, L * EMBED)])
    pltpu.sync_copy(pos_hbm, pos_v.at[pl.ds(L * EMBED, L * EMBED)])

    gbuf = (g0, g1)
    obuf = (o0, o1)
    gsem = (gs0, gs1)
    wsem = (ws0, ws1)

    def gather_copy(cc, b):
        return pltpu.make_async_copy(wE_hbm.at[idx_v.at[cc]], gbuf[b], gsem[b])

    def write_copy(cc, b):
        return pltpu.make_async_copy(obuf[b], out_hbm.at[base + cc], wsem[b])

    def chunk_step(cc, b, start):
        gather_copy(cc, b).wait()

        @pl.when(cc >= 2)
        def _():
            write_copy(cc - 2, b).wait()

        pbase = start * EMBED

        def add_row(j, carry):
            for k in range(EMBED // 16):
                sl = pl.ds(16 * k, 16)
                obuf[b][j, sl] = (gbuf[b][j, sl]
                                  + pos_v[pl.ds(pbase + j * EMBED + 16 * k, 16)])
            return carry

        lax.fori_loop(0, LANES, add_row, 0)

        @pl.when(cc + 2 < CPW)
        def _():
            gather_copy(cc + 2, b).start()

        write_copy(cc, b).start()

    gather_copy(0, 0).start()
    gather_copy(1, 1).start()

    def loop_body(i, start):
        chunk_step(2 * i, 0, start)
        start1 = start + LANES - L
        start1 = start1 + jnp.where(start1 < 0, L, 0)
        chunk_step(2 * i + 1, 1, start1)
        start2 = start1 + LANES - L
        start2 = start2 + jnp.where(start2 < 0, L, 0)
        return start2

    lax.fori_loop(0, CPW // 2, loop_body, jnp.int32(0))

    write_copy(CPW - 2, 0).wait()
    write_copy(CPW - 1, 1).wait()


def _sc_embed(tok, W_E_wide, W_pos_flat):
    mesh = plsc.VectorSubcoreMesh(core_axis_name="c", subcore_axis_name="s")
    kern = functools.partial(
        pl.kernel,
        out_type=jax.ShapeDtypeStruct((NIDX, LANES, EMBED), jnp.float32),
        mesh=mesh,
        scratch_types=[
            pltpu.VMEM((CPW, LANES), jnp.int32),
            pltpu.VMEM((2 * L * EMBED,), jnp.float32),
            pltpu.VMEM((LANES, LANES), jnp.float32),
            pltpu.VMEM((LANES, LANES), jnp.float32),
            pltpu.VMEM((LANES, EMBED), jnp.float32),
            pltpu.VMEM((LANES, EMBED), jnp.float32),
            pltpu.SemaphoreType.DMA,
            pltpu.SemaphoreType.DMA,
            pltpu.SemaphoreType.DMA,
            pltpu.SemaphoreType.DMA,
        ],
        compiler_params=pltpu.CompilerParams(use_tc_tiling_on_sc=True),
    )(_worker_body)
    return kern(tok, W_E_wide, W_pos_flat)


def kernel(tokens, W_E, W_pos):
    tok = tokens.reshape(NIDX, LANES).astype(jnp.int32)
    wide = _widen(W_E)
    out = _sc_embed(tok, wide, W_pos.reshape(L * EMBED))
    return out.reshape(B, L, EMBED)


# speedup vs baseline: 1.9488x; 1.1477x over previous
import functools

import jax
import jax.numpy as jnp
from jax import lax
from jax.experimental import pallas as pl
from jax.experimental.pallas import tpu as pltpu
from jax.experimental.pallas import tpu_sc as plsc

VOCAB = 1000000
EMBED = 64
B = 4096
L = 200
LANES = 128

NC = 2
NS = 16
NW = NC * NS
NIDX = (B * L) // LANES
CPW = NIDX // NW

A_BLK = 8192
A_GRID = (VOCAB + A_BLK - 1) // A_BLK


def _widen_body(wt_ref, wide_ref):
    eye = jnp.eye(EMBED, LANES, dtype=jnp.float32)
    wide_ref[...] = jax.lax.dot_general(
        wt_ref[...], eye, (((0,), (0,)), ((), ())),
        preferred_element_type=jnp.float32)


def _widen(W_E):
    wt = W_E.T
    return pl.pallas_call(
        _widen_body,
        grid=(A_GRID,),
        in_specs=[pl.BlockSpec((EMBED, A_BLK), lambda i: (0, i))],
        out_specs=pl.BlockSpec((A_BLK, LANES), lambda i: (i, 0)),
        out_shape=jax.ShapeDtypeStruct((VOCAB, LANES), jnp.float32),
        compiler_params=pltpu.CompilerParams(
            dimension_semantics=("arbitrary",)),
    )(wt)


def _worker_body(tok_hbm, wE_hbm, pos_hbm, out_hbm,
                 idx_v, pos_v, g0, g1, g2, o0, o1,
                 gs0, gs1, gs2, ws0, ws1):
    cid = lax.axis_index("c")
    sid = lax.axis_index("s")
    wid = sid * NC + cid
    base = wid * CPW

    pltpu.sync_copy(tok_hbm.at[pl.ds(base, CPW)], idx_v)
    pltpu.sync_copy(pos_hbm, pos_v)

    gbuf = (g0, g1, g2)
    gsem = (gs0, gs1, gs2)
    obuf = (o0, o1)
    wsem = (ws0, ws1)

    def gather_copy(cc, gb):
        return pltpu.make_async_copy(wE_hbm.at[idx_v.at[cc]], gbuf[gb],
                                     gsem[gb])

    def write_copy(cc, ob):
        return pltpu.make_async_copy(obuf[ob], out_hbm.at[base + cc],
                                     wsem[ob])

    def chunk_step(cc, gb, ob, start):
        gather_copy(cc, gb).wait()

        @pl.when(cc >= 2)
        def _():
            write_copy(cc - 2, ob).wait()

        split = L - start

        def add_lo(j, carry):
            pb = (start + j) * EMBED
            for k in range(EMBED // 16):
                sl = pl.ds(16 * k, 16)
                obuf[ob][j, sl] = (gbuf[gb][j, sl]
                                   + pos_v[pl.ds(pb + 16 * k, 16)])
            return carry

        def add_hi(j, carry):
            pb = (start + j - L) * EMBED
            for k in range(EMBED // 16):
                sl = pl.ds(16 * k, 16)
                obuf[ob][j, sl] = (gbuf[gb][j, sl]
                                   + pos_v[pl.ds(pb + 16 * k, 16)])
            return carry

        lax.fori_loop(0, lax.min(split, LANES), add_lo, 0)
        lax.fori_loop(lax.min(split, LANES), LANES, add_hi, 0)

        @pl.when(cc + 3 < CPW)
        def _():
            gather_copy(cc + 3, gb).start()

        write_copy(cc, ob).start()

    gather_copy(0, 0).start()
    gather_copy(1, 1).start()
    gather_copy(2, 2).start()

    def loop_body(i, start):
        for u in range(6):
            cc = 6 * i + u
            chunk_step(cc, u % 3, u % 2, start)
            start = start + LANES - L
            start = start + jnp.where(start < 0, L, 0)
        return start

    start = lax.fori_loop(0, CPW // 6, loop_body, jnp.int32(0))
    chunk_step(CPW - 2, (CPW - 2) % 3, 0, start)
    start = start + LANES - L
    start = start + jnp.where(start < 0, L, 0)
    chunk_step(CPW - 1, (CPW - 1) % 3, 1, start)

    write_copy(CPW - 2, 0).wait()
    write_copy(CPW - 1, 1).wait()


def _sc_embed(tok, W_E_wide, W_pos_flat):
    mesh = plsc.VectorSubcoreMesh(core_axis_name="c", subcore_axis_name="s")
    kern = functools.partial(
        pl.kernel,
        out_type=jax.ShapeDtypeStruct((NIDX, LANES, EMBED), jnp.float32),
        mesh=mesh,
        scratch_types=[
            pltpu.VMEM((CPW, LANES), jnp.int32),
            pltpu.VMEM((L * EMBED,), jnp.float32),
            pltpu.VMEM((LANES, LANES), jnp.float32),
            pltpu.VMEM((LANES, LANES), jnp.float32),
            pltpu.VMEM((LANES, LANES), jnp.float32),
            pltpu.VMEM((LANES, EMBED), jnp.float32),
            pltpu.VMEM((LANES, EMBED), jnp.float32),
            pltpu.SemaphoreType.DMA,
            pltpu.SemaphoreType.DMA,
            pltpu.SemaphoreType.DMA,
            pltpu.SemaphoreType.DMA,
            pltpu.SemaphoreType.DMA,
        ],
        compiler_params=pltpu.CompilerParams(use_tc_tiling_on_sc=True),
    )(_worker_body)
    return kern(tok, W_E_wide, W_pos_flat)


def kernel(tokens, W_E, W_pos):
    tok = tokens.reshape(NIDX, LANES).astype(jnp.int32)
    wide = _widen(W_E)
    out = _sc_embed(tok, wide, W_pos.reshape(L * EMBED))
    return out.reshape(B, L, EMBED)


# speedup vs baseline: 2.0689x; 1.0616x over previous
import functools

import jax
import jax.numpy as jnp
from jax import lax
from jax.experimental import pallas as pl
from jax.experimental.pallas import tpu as pltpu
from jax.experimental.pallas import tpu_sc as plsc

VOCAB = 1000000
EMBED = 64
B = 4096
L = 200
LANES = 128

NC = 2
NS = 16
NW = NC * NS
NIDX = (B * L) // LANES
CPW = NIDX // NW

A_BLK = 8192
A_GRID = (VOCAB + A_BLK - 1) // A_BLK


def _widen_body(wt_ref, wide_ref):
    eye = jnp.eye(EMBED, LANES, dtype=jnp.float32)
    wide_ref[...] = jax.lax.dot_general(
        wt_ref[...], eye, (((0,), (0,)), ((), ())),
        preferred_element_type=jnp.float32)


def _widen(W_E):
    wt = W_E.T
    return pl.pallas_call(
        _widen_body,
        grid=(A_GRID,),
        in_specs=[pl.BlockSpec((EMBED, A_BLK), lambda i: (0, i))],
        out_specs=pl.BlockSpec((A_BLK, LANES), lambda i: (i, 0)),
        out_shape=jax.ShapeDtypeStruct((VOCAB, LANES), jnp.float32),
        compiler_params=pltpu.CompilerParams(
            dimension_semantics=("arbitrary",)),
    )(wt)


def _worker_body(tok_hbm, wE_hbm, pos_hbm, out_hbm,
                 idx_v, pos_v, g0, g1, g2, o0, o1,
                 gs0, gs1, gs2, ws0, ws1):
    cid = lax.axis_index("c")
    sid = lax.axis_index("s")
    wid = sid * NC + cid
    base = wid * CPW

    pltpu.sync_copy(tok_hbm.at[pl.ds(base, CPW)], idx_v)
    pltpu.sync_copy(pos_hbm, pos_v.at[pl.ds(0, L * EMBED)])
    pltpu.sync_copy(pos_hbm.at[pl.ds(0, LANES * EMBED)],
                    pos_v.at[pl.ds(L * EMBED, LANES * EMBED)])

    gbuf = (g0, g1, g2)
    gsem = (gs0, gs1, gs2)
    obuf = (o0, o1)
    wsem = (ws0, ws1)

    def gather_copy(cc, gb):
        return pltpu.make_async_copy(wE_hbm.at[idx_v.at[cc]], gbuf[gb],
                                     gsem[gb])

    def write_copy(cc, ob):
        return pltpu.make_async_copy(obuf[ob], out_hbm.at[base + cc],
                                     wsem[ob])

    def chunk_step(cc, gb, ob, start):
        gather_copy(cc, gb).wait()

        @pl.when(cc >= 2)
        def _():
            write_copy(cc - 2, ob).wait()

        pbase = start * EMBED

        def add_rows(i, carry):
            j0 = 4 * i
            pb = pbase + j0 * EMBED
            for r in range(4):
                for k in range(EMBED // 16):
                    sl = pl.ds(16 * k, 16)
                    obuf[ob][j0 + r, sl] = (
                        gbuf[gb][j0 + r, sl]
                        + pos_v[pl.ds(pb + r * EMBED + 16 * k, 16)])
            return carry

        lax.fori_loop(0, LANES // 4, add_rows, 0)

        @pl.when(cc + 3 < CPW)
        def _():
            gather_copy(cc + 3, gb).start()

        write_copy(cc, ob).start()

    gather_copy(0, 0).start()
    gather_copy(1, 1).start()
    gather_copy(2, 2).start()

    def loop_body(i, start):
        for u in range(6):
            cc = 6 * i + u
            chunk_step(cc, u % 3, u % 2, start)
            start = start + LANES - L
            start = start + jnp.where(start < 0, L, 0)
        return start

    start = lax.fori_loop(0, CPW // 6, loop_body, jnp.int32(0))
    chunk_step(CPW - 2, (CPW - 2) % 3, 0, start)
    start = start + LANES - L
    start = start + jnp.where(start < 0, L, 0)
    chunk_step(CPW - 1, (CPW - 1) % 3, 1, start)

    write_copy(CPW - 2, 0).wait()
    write_copy(CPW - 1, 1).wait()


def _sc_embed(tok, W_E_wide, W_pos_flat):
    mesh = plsc.VectorSubcoreMesh(core_axis_name="c", subcore_axis_name="s")
    kern = functools.partial(
        pl.kernel,
        out_type=jax.ShapeDtypeStruct((NIDX, LANES, EMBED), jnp.float32),
        mesh=mesh,
        scratch_types=[
            pltpu.VMEM((CPW, LANES), jnp.int32),
            pltpu.VMEM(((L + LANES) * EMBED,), jnp.float32),
            pltpu.VMEM((LANES, LANES), jnp.float32),
            pltpu.VMEM((LANES, LANES), jnp.float32),
            pltpu.VMEM((LANES, LANES), jnp.float32),
            pltpu.VMEM((LANES, EMBED), jnp.float32),
            pltpu.VMEM((LANES, EMBED), jnp.float32),
            pltpu.SemaphoreType.DMA,
            pltpu.SemaphoreType.DMA,
            pltpu.SemaphoreType.DMA,
            pltpu.SemaphoreType.DMA,
            pltpu.SemaphoreType.DMA,
        ],
        compiler_params=pltpu.CompilerParams(use_tc_tiling_on_sc=True),
    )(_worker_body)
    return kern(tok, W_E_wide, W_pos_flat)


def kernel(tokens, W_E, W_pos):
    tok = tokens.reshape(NIDX, LANES).astype(jnp.int32)
    wide = _widen(W_E)
    out = _sc_embed(tok, wide, W_pos.reshape(L * EMBED))
    return out.reshape(B, L, EMBED)
